# baseline (speedup 1.0000x reference)
import functools

import jax
import jax.numpy as jnp
from jax import lax
from jax.experimental import pallas as pl
from jax.experimental.pallas import tpu as pltpu
from jax.experimental.pallas import tpu_sc as plsc

B_TOK = 16384
SEQ = 26
NUM_ROWS = 1000000
DIM = 64

NC = 2
NS = 16
NW = NC * NS
CH = 128
B = B_TOK * SEQ
NCHUNK = B // CH
CPW = NCHUNK // NW
NBUF = 4
NGRP = CPW // NBUF


def _build_gather():
    mesh = plsc.VectorSubcoreMesh(core_axis_name="c", subcore_axis_name="s")

    @functools.partial(
        pl.kernel,
        out_type=jax.ShapeDtypeStruct((B, DIM), jnp.float32),
        mesh=mesh,
        compiler_params=pltpu.CompilerParams(use_tc_tiling_on_sc=False),
        scratch_types=[
            pltpu.VMEM((CPW, CH), jnp.int32),
            pltpu.VMEM((NBUF, CH, DIM), jnp.float32),
            pltpu.SemaphoreType.DMA,
        ],
    )
    def grab(table_hbm, idx_hbm, out_hbm, idx_v, rows_v, gsem):
        wid = lax.axis_index("s") * NC + lax.axis_index("c")
        chunk0 = wid * CPW
        pltpu.sync_copy(idx_hbm.at[pl.ds(chunk0, CPW)], idx_v)

        def fire(j, b):
            pltpu.async_copy(table_hbm.at[idx_v.at[j]], rows_v.at[b], gsem)

        for b in range(NBUF):
            fire(b, b)

        def group(g, carry):
            for b in range(NBUF):
                j = g * NBUF + b
                pltpu.make_async_copy(
                    table_hbm.at[idx_v.at[j]], rows_v.at[b], gsem
                ).wait()
                pltpu.sync_copy(
                    rows_v.at[b], out_hbm.at[pl.ds((chunk0 + j) * CH, CH)]
                )

                @pl.when(j + NBUF < CPW)
                def _fire_next():
                    fire(j + NBUF, b)

            return carry

        lax.fori_loop(0, NGRP, group, 0)

    return grab


def kernel(token_ids, embedding_table):
    idx = token_ids.astype(jnp.int32).reshape(NCHUNK, CH)
    out = _build_gather()(embedding_table, idx)
    return out.reshape(B_TOK, SEQ, DIM)
